# baseline (speedup 1.0000x reference)
import functools

import jax
import jax.numpy as jnp
from jax import lax
from jax.experimental import pallas as pl
from jax.experimental.pallas import tpu as pltpu
from jax.experimental.pallas import tpu_sc as plsc

_BATCH = 4096
_HIST = 200
_DIM = 32
_VOCAB = 1000000


def _transpose_kernel():
    info = plsc.get_sparse_core_info()
    nw = info.num_cores * info.num_subcores
    n_slabs = _VOCAB // 128
    mesh = plsc.VectorSubcoreMesh(core_axis_name="c", subcore_axis_name="s")

    per_w = -(-n_slabs // nw)
    per_w += (-per_w) % 4

    @functools.partial(
        pl.kernel,
        mesh=mesh,
        out_type=jax.ShapeDtypeStruct((_VOCAB // 4, 128), jnp.float32),
        scratch_types=[
            pltpu.VMEM((4, _DIM, 128), jnp.float32),
            pltpu.VMEM((4, _DIM, 128), jnp.float32),
            pltpu.SemaphoreType.DMA,
            pltpu.SemaphoreType.DMA,
            pltpu.SemaphoreType.DMA,
            pltpu.SemaphoreType.DMA,
            pltpu.SemaphoreType.DMA,
        ],
        compiler_params=pltpu.CompilerParams(needs_layout_passes=False),
    )
    def k1(tab_hbm, tail_hbm, scr_hbm, i_v, o_v, *sems):
        wid = lax.axis_index("s") * info.num_cores + lax.axis_index("c")
        iota = lax.iota(jnp.int32, 16)
        isems = sems[:4]
        so = sems[4]
        base = wid * per_w

        def slab_of(j):
            return jnp.minimum(base + j, n_slabs - 1)

        def fetch(j, b):
            pltpu.async_copy(
                tab_hbm.at[:, pl.ds(128 * slab_of(j), 128)],
                i_v.at[b],
                isems[b],
            )

        for b in range(3):
            fetch(b, b)

        def quad(g, carry):
            for b in range(4):
                j = g * 4 + b
                pltpu.make_async_copy(
                    tab_hbm.at[:, pl.ds(0, 128)], i_v.at[b], isems[b]
                ).wait()

                @pl.when(j + 3 < per_w)
                def _():
                    fetch(j + 3, (b + 3) % 4)

                @pl.when(j >= 4)
                def _():
                    pltpu.make_async_copy(
                        o_v.at[b], scr_hbm.at[pl.ds(0, _DIM), :], so
                    ).wait()

                for dd in range(2):
                    for ll in range(8):
                        a = [
                            i_v[b, 16 * dd + jj, pl.ds(16 * ll, 16)]
                            for jj in range(16)
                        ]
                        for st in (1, 2, 4, 8):
                            ixs = iota ^ st
                            mks = (iota & st) == 0
                            na = []
                            for r in range(16):
                                p = a[r ^ st].at[ixs].get(
                                    mode="promise_in_bounds"
                                )
                                if r & st == 0:
                                    na.append(jnp.where(mks, a[r], p))
                                else:
                                    na.append(jnp.where(mks, p, a[r]))
                            a = na
                        for i in range(16):
                            ln = 16 * ll + i
                            o_v[
                                b,
                                ln // 4,
                                pl.ds(32 * (ln % 4) + 16 * dd, 16),
                            ] = a[i]

                pltpu.async_copy(
                    o_v.at[b],
                    scr_hbm.at[pl.ds(_DIM * slab_of(j), _DIM), :],
                    so,
                )
            return carry

        lax.fori_loop(0, per_w // 4, quad, 0)
        for b in range(4):
            pltpu.make_async_copy(
                o_v.at[b], scr_hbm.at[pl.ds(0, _DIM), :], so
            ).wait()

        @pl.when(wid == 0)
        def _():
            pltpu.sync_copy(tail_hbm, i_v.at[0, pl.ds(0, 16)])
            pltpu.sync_copy(
                i_v.at[0, pl.ds(0, 16)],
                scr_hbm.at[pl.ds(_DIM * n_slabs, 16), :],
            )

    return k1


def _gather_kernel():
    info = plsc.get_sparse_core_info()
    nw = info.num_cores * info.num_subcores
    n_units = (_HIST // 8) * (_BATCH // 128)
    per_w = n_units // nw
    bt_n = _BATCH // 128
    mesh = plsc.VectorSubcoreMesh(core_axis_name="c", subcore_axis_name="s")

    @functools.partial(
        pl.kernel,
        mesh=mesh,
        out_type=jax.ShapeDtypeStruct((_HIST, _DIM, _BATCH), jnp.float32),
        scratch_types=[
            pltpu.VMEM((8, 128), jnp.int32),
            pltpu.VMEM((8, 128), jnp.int32),
            pltpu.VMEM((8, 128), jnp.int32),
            pltpu.VMEM((4, 128, 128), jnp.float32),
            pltpu.VMEM((2, _DIM, 128), jnp.float32),
            pltpu.SemaphoreType.DMA,
            pltpu.SemaphoreType.DMA,
            pltpu.SemaphoreType.DMA,
            pltpu.SemaphoreType.DMA,
            pltpu.SemaphoreType.DMA,
        ],
        compiler_params=pltpu.CompilerParams(needs_layout_passes=False),
    )
    def k2(
        scr_hbm, idx_hbm, out_hbm, idx_v, p_v, q_v, g_v, o_v, *sems
    ):
        wid = lax.axis_index("s") * info.num_cores + lax.axis_index("c")
        iota = lax.iota(jnp.int32, 16)
        gsems = sems[:4]
        so = sems[4]

        def unit(kk, carry):
            u = wid * per_w + kk
            lt = u // bt_n
            bt = u % bt_n
            bofs = bt * 128

            pltpu.sync_copy(
                idx_hbm.at[pl.ds(lt * 8, 8), pl.ds(bofs, 128)], idx_v
            )
            for s2 in range(8):
                for j in range(8):
                    v = idx_v[s2, pl.ds(16 * j, 16)]
                    p_v[s2, pl.ds(16 * j, 16)] = lax.shift_right_logical(v, 2)
                    q_v[s2, pl.ds(16 * j, 16)] = lax.shift_left(v & 3, 5)

            for s in range(3):
                pltpu.async_copy(scr_hbm.at[p_v.at[s]], g_v.at[s], gsems[s])
            for s in range(8):
                bg = s % 4
                b = s % 2
                pltpu.make_async_copy(
                    scr_hbm.at[p_v.at[s]], g_v.at[bg], gsems[bg]
                ).wait()
                if s + 3 < 8:
                    pltpu.async_copy(
                        scr_hbm.at[p_v.at[s + 3]],
                        g_v.at[(s + 3) % 4],
                        gsems[(s + 3) % 4],
                    )
                if s >= 2:
                    pltpu.make_async_copy(
                        o_v.at[b], out_hbm.at[0, :, pl.ds(0, 128)], so
                    ).wait()
                else:

                    @pl.when(kk > 0)
                    def _():
                        pltpu.make_async_copy(
                            o_v.at[b], out_hbm.at[0, :, pl.ds(0, 128)], so
                        ).wait()

                def blk(t, carry, _s=s, _b=b, _bg=bg):
                    brow = 16 * (t // 2)
                    dof = 16 * (t % 2)
                    qvec = q_v[_s, pl.ds(brow, 16)] + dof
                    a = []
                    for jj in range(16):
                        a.append(g_v[_bg, brow + jj, pl.ds(qvec[jj], 16)])
                    for st in (1, 2, 4, 8):
                        ixs = iota ^ st
                        mks = (iota & st) == 0
                        na = []
                        for r in range(16):
                            p = a[r ^ st].at[ixs].get(
                                mode="promise_in_bounds"
                            )
                            if r & st == 0:
                                na.append(jnp.where(mks, a[r], p))
                            else:
                                na.append(jnp.where(mks, p, a[r]))
                        a = na
                    for i in range(16):
                        o_v[_b, dof + i, pl.ds(brow, 16)] = a[i]
                    return carry

                lax.fori_loop(0, 16, blk, 0)
                pltpu.async_copy(
                    o_v.at[b], out_hbm.at[lt * 8 + s, :, pl.ds(bofs, 128)], so
                )
            return carry

        lax.fori_loop(0, per_w, unit, 0)
        for b in range(2):
            pltpu.make_async_copy(
                o_v.at[b], out_hbm.at[0, :, pl.ds(0, 128)], so
            ).wait()

    return k2


_transpose = _transpose_kernel()
_gather = _gather_kernel()


@jax.jit
def kernel(indices, table):
    n_full = (_VOCAB // 128) * 128
    tail = lax.slice(table, (n_full, 0), (_VOCAB, _DIM)).reshape(16, 128)
    scr = _transpose(table.T, tail)
    idx_t = indices.astype(jnp.int32).T
    out = _gather(scr, idx_t)
    return out.transpose(2, 0, 1)


# speedup vs baseline: 1.0395x; 1.0395x over previous
import functools

import jax
import jax.numpy as jnp
from jax import lax
from jax.experimental import pallas as pl
from jax.experimental.pallas import tpu as pltpu
from jax.experimental.pallas import tpu_sc as plsc

_BATCH = 4096
_HIST = 200
_DIM = 32
_VOCAB = 1000000


def _transpose_kernel():
    info = plsc.get_sparse_core_info()
    nw = info.num_cores * info.num_subcores
    n_slabs = _VOCAB // 128
    mesh = plsc.VectorSubcoreMesh(core_axis_name="c", subcore_axis_name="s")

    per_w = -(-n_slabs // nw)
    per_w += per_w % 2

    @functools.partial(
        pl.kernel,
        mesh=mesh,
        out_type=jax.ShapeDtypeStruct((_VOCAB // 4, 128), jnp.float32),
        scratch_types=[
            pltpu.VMEM((2, _DIM, 128), jnp.float32),
            pltpu.VMEM((2, _DIM, 128), jnp.float32),
            pltpu.SemaphoreType.DMA,
            pltpu.SemaphoreType.DMA,
            pltpu.SemaphoreType.DMA,
        ],
        compiler_params=pltpu.CompilerParams(needs_layout_passes=False),
    )
    def k1(tab_hbm, tail_hbm, scr_hbm, i_v, o_v, si0, si1, so):
        wid = lax.axis_index("s") * info.num_cores + lax.axis_index("c")
        iota = lax.iota(jnp.int32, 16)
        isems = (si0, si1)
        base = wid * per_w

        def slab_of(j):
            return jnp.minimum(base + j, n_slabs - 1)

        def fetch(j, b):
            pltpu.async_copy(
                tab_hbm.at[:, pl.ds(128 * slab_of(j), 128)],
                i_v.at[b],
                isems[b],
            )

        fetch(0, 0)

        def pair(g, carry):
            for b in range(2):
                j = g * 2 + b
                pltpu.make_async_copy(
                    tab_hbm.at[:, pl.ds(0, 128)], i_v.at[b], isems[b]
                ).wait()

                @pl.when(j + 1 < per_w)
                def _():
                    fetch(j + 1, 1 - b)

                @pl.when(j >= 2)
                def _():
                    pltpu.make_async_copy(
                        o_v.at[b], scr_hbm.at[pl.ds(0, _DIM), :], so
                    ).wait()

                for dd in range(2):
                    for ll in range(8):
                        a = [
                            i_v[b, 16 * dd + jj, pl.ds(16 * ll, 16)]
                            for jj in range(16)
                        ]
                        for st in (1, 2, 4, 8):
                            ixs = iota ^ st
                            mks = (iota & st) == 0
                            na = []
                            for r in range(16):
                                p = a[r ^ st].at[ixs].get(
                                    mode="promise_in_bounds"
                                )
                                if r & st == 0:
                                    na.append(jnp.where(mks, a[r], p))
                                else:
                                    na.append(jnp.where(mks, p, a[r]))
                            a = na
                        for i in range(16):
                            ln = 16 * ll + i
                            o_v[
                                b,
                                ln // 4,
                                pl.ds(32 * (ln % 4) + 16 * dd, 16),
                            ] = a[i]

                pltpu.async_copy(
                    o_v.at[b],
                    scr_hbm.at[pl.ds(_DIM * slab_of(j), _DIM), :],
                    so,
                )
            return carry

        lax.fori_loop(0, per_w // 2, pair, 0)
        for b in range(2):
            pltpu.make_async_copy(
                o_v.at[b], scr_hbm.at[pl.ds(0, _DIM), :], so
            ).wait()

        @pl.when(wid == 0)
        def _():
            pltpu.sync_copy(tail_hbm, i_v.at[0, pl.ds(0, 16)])
            pltpu.sync_copy(
                i_v.at[0, pl.ds(0, 16)],
                scr_hbm.at[pl.ds(_DIM * n_slabs, 16), :],
            )

    return k1


def _gather_kernel():
    info = plsc.get_sparse_core_info()
    nw = info.num_cores * info.num_subcores
    n_units = (_HIST // 8) * (_BATCH // 128)
    per_w = n_units // nw
    bt_n = _BATCH // 128
    mesh = plsc.VectorSubcoreMesh(core_axis_name="c", subcore_axis_name="s")

    @functools.partial(
        pl.kernel,
        mesh=mesh,
        out_type=jax.ShapeDtypeStruct((_HIST, _DIM, _BATCH), jnp.float32),
        scratch_types=[
            pltpu.VMEM((8, 128), jnp.int32),
            pltpu.VMEM((8, 128), jnp.int32),
            pltpu.VMEM((8, 128), jnp.int32),
            pltpu.VMEM((4, 128, 128), jnp.float32),
            pltpu.VMEM((2, _DIM, 128), jnp.float32),
            pltpu.SemaphoreType.DMA,
            pltpu.SemaphoreType.DMA,
            pltpu.SemaphoreType.DMA,
            pltpu.SemaphoreType.DMA,
            pltpu.SemaphoreType.DMA,
        ],
        compiler_params=pltpu.CompilerParams(needs_layout_passes=False),
    )
    def k2(
        scr_hbm, idx_hbm, out_hbm, idx_v, p_v, q_v, g_v, o_v, *sems
    ):
        wid = lax.axis_index("s") * info.num_cores + lax.axis_index("c")
        iota = lax.iota(jnp.int32, 16)
        gsems = sems[:4]
        so = sems[4]

        def unit(kk, carry):
            u = wid * per_w + kk
            lt = u // bt_n
            bt = u % bt_n
            bofs = bt * 128

            pltpu.sync_copy(
                idx_hbm.at[pl.ds(lt * 8, 8), pl.ds(bofs, 128)], idx_v
            )
            for s2 in range(8):
                for j in range(8):
                    v = idx_v[s2, pl.ds(16 * j, 16)]
                    p_v[s2, pl.ds(16 * j, 16)] = lax.shift_right_logical(v, 2)
                    q_v[s2, pl.ds(16 * j, 16)] = lax.shift_left(v & 3, 5)

            for s in range(3):
                pltpu.async_copy(scr_hbm.at[p_v.at[s]], g_v.at[s], gsems[s])
            for s in range(8):
                bg = s % 4
                b = s % 2
                pltpu.make_async_copy(
                    scr_hbm.at[p_v.at[s]], g_v.at[bg], gsems[bg]
                ).wait()
                if s + 3 < 8:
                    pltpu.async_copy(
                        scr_hbm.at[p_v.at[s + 3]],
                        g_v.at[(s + 3) % 4],
                        gsems[(s + 3) % 4],
                    )
                if s >= 2:
                    pltpu.make_async_copy(
                        o_v.at[b], out_hbm.at[0, :, pl.ds(0, 128)], so
                    ).wait()
                else:

                    @pl.when(kk > 0)
                    def _():
                        pltpu.make_async_copy(
                            o_v.at[b], out_hbm.at[0, :, pl.ds(0, 128)], so
                        ).wait()

                def blk(t, carry, _s=s, _b=b, _bg=bg):
                    brow = 16 * (t // 2)
                    dof = 16 * (t % 2)
                    qvec = q_v[_s, pl.ds(brow, 16)] + dof
                    a = []
                    for jj in range(16):
                        a.append(g_v[_bg, brow + jj, pl.ds(qvec[jj], 16)])
                    for st in (1, 2, 4, 8):
                        ixs = iota ^ st
                        mks = (iota & st) == 0
                        na = []
                        for r in range(16):
                            p = a[r ^ st].at[ixs].get(
                                mode="promise_in_bounds"
                            )
                            if r & st == 0:
                                na.append(jnp.where(mks, a[r], p))
                            else:
                                na.append(jnp.where(mks, p, a[r]))
                        a = na
                    for i in range(16):
                        o_v[_b, dof + i, pl.ds(brow, 16)] = a[i]
                    return carry

                lax.fori_loop(0, 16, blk, 0)
                pltpu.async_copy(
                    o_v.at[b], out_hbm.at[lt * 8 + s, :, pl.ds(bofs, 128)], so
                )
            return carry

        lax.fori_loop(0, per_w, unit, 0)
        for b in range(2):
            pltpu.make_async_copy(
                o_v.at[b], out_hbm.at[0, :, pl.ds(0, 128)], so
            ).wait()

    return k2


_transpose = _transpose_kernel()
_gather = _gather_kernel()


@jax.jit
def kernel(indices, table):
    n_full = (_VOCAB // 128) * 128
    tail = lax.slice(table, (n_full, 0), (_VOCAB, _DIM)).reshape(16, 128)
    scr = _transpose(table.T, tail)
    idx_t = indices.astype(jnp.int32).T
    out = _gather(scr, idx_t)
    return out.transpose(2, 0, 1)
